# Initial kernel scaffold; baseline (speedup 1.0000x reference)
import functools

import jax
import jax.numpy as jnp
from jax import lax
from jax.experimental import pallas as pl
from jax.experimental.pallas import tpu as pltpu
from jax.experimental.pallas import tpu_sc as plsc

NC = 2
NS = 16
L = 16
IB = 128
CHUNK = 1024
KI = CHUNK // IB


def _round_up(x, m):
    return (x + m - 1) // m * m


def _phase_a(N, EP):
    ea = EP // (NC * NS)
    nchunks = ea // CHUNK
    rows_sub = N // NS
    nz = 10
    zb = rows_sub // nz

    mesh = plsc.VectorSubcoreMesh(core_axis_name="c", subcore_axis_name="s")

    @functools.partial(
        pl.kernel,
        out_type=jax.ShapeDtypeStruct((NC, N, 16), jnp.float32),
        mesh=mesh,
        scratch_types=[
            pltpu.VMEM((KI, IB), jnp.int32),
            pltpu.VMEM((KI, IB), jnp.int32),
            pltpu.VMEM((CHUNK, 16), jnp.float32),
            pltpu.VMEM((CHUNK, 16), jnp.float32),
            pltpu.VMEM((625, 16), jnp.float32),
            pltpu.VMEM_SHARED((N, 16), jnp.float32),
            pltpu.SemaphoreType.DMA,
        ],
    )
    def phase_a(src_hbm, dst_hbm, pe_hbm, nf_hbm, out_hbm,
                sidx, didx, pe_v, rows_v, zb_v, acc_sh, sem):
        cid = lax.axis_index("c")
        sid = lax.axis_index("s")
        wid = sid * NC + cid

        @pl.loop(0, zb)
        def _(i):
            zb_v[i, :] = jnp.zeros((16,), jnp.float32)

        row0 = sid * rows_sub

        @pl.loop(0, nz)
        def _(k):
            pltpu.sync_copy(zb_v.at[pl.ds(0, zb), :],
                            acc_sh.at[pl.ds(row0 + k * zb, zb), :])

        plsc.subcore_barrier()

        base0 = wid * ea

        @pl.loop(0, nchunks)
        def _(ci):
            base = base0 + ci * CHUNK
            cbase = base // IB
            pltpu.sync_copy(src_hbm.at[pl.ds(cbase, KI), :], sidx)
            pltpu.sync_copy(dst_hbm.at[pl.ds(cbase, KI), :], didx)
            pltpu.sync_copy(pe_hbm.at[pl.ds(base, CHUNK), :], pe_v)
            cps = [
                pltpu.async_copy(nf_hbm.at[sidx.at[j]],
                                 rows_v.at[pl.ds(j * IB, IB), :], sem)
                for j in range(KI)
            ]
            for cp in cps:
                cp.wait()

            @pl.loop(0, CHUNK)
            def _(i):
                mrow = jnp.full((L,), i, jnp.int32)
                mcol = jnp.full((L,), 15, jnp.int32)
                mb = plsc.load_gather(pe_v, [mrow, mcol])
                rows_v[i, :] = mb * rows_v[i, :] + pe_v[i, :]

            for j in range(KI):
                pltpu.sync_copy(rows_v.at[pl.ds(j * IB, IB), :],
                                acc_sh.at[didx.at[j]], add=True)

        plsc.subcore_barrier()
        pltpu.sync_copy(acc_sh.at[pl.ds(row0, rows_sub), :],
                        out_hbm.at[cid, pl.ds(row0, rows_sub), :])

    return phase_a


def _phase_b(N, EP):
    eb = EP // NS
    nchunks = eb // CHUNK
    rows_sub = N // NS
    nz = 10
    zb = rows_sub // nz

    mesh = plsc.VectorSubcoreMesh(core_axis_name="c", subcore_axis_name="s")

    @functools.partial(
        pl.kernel,
        out_type=jax.ShapeDtypeStruct((NC, N, 16), jnp.float32),
        mesh=mesh,
        scratch_types=[
            pltpu.VMEM((KI, IB), jnp.int32),
            pltpu.VMEM((KI, IB), jnp.int32),
            pltpu.VMEM((CHUNK,), jnp.float32),
            pltpu.VMEM((CHUNK, 16), jnp.float32),
            pltpu.VMEM((625, 16), jnp.float32),
            pltpu.VMEM_SHARED((N, 16), jnp.float32),
            pltpu.SemaphoreType.DMA,
        ],
    )
    def phase_b(src_hbm, dst_hbm, mask_hbm, h_hbm, out_hbm,
                sidx, didx, mask_v, rows_v, zb_v, acc_sh, sem):
        cid = lax.axis_index("c")
        sid = lax.axis_index("s")

        @pl.loop(0, zb)
        def _(i):
            zb_v[i, :] = jnp.zeros((16,), jnp.float32)

        row0 = sid * rows_sub

        @pl.loop(0, nz)
        def _(k):
            pltpu.sync_copy(zb_v.at[pl.ds(0, zb), :],
                            acc_sh.at[pl.ds(row0 + k * zb, zb), :])

        plsc.subcore_barrier()

        base0 = sid * eb
        goff = cid * N

        @pl.loop(0, nchunks)
        def _(ci):
            base = base0 + ci * CHUNK
            cbase = base // IB
            pltpu.sync_copy(src_hbm.at[pl.ds(cbase, KI), :], sidx)
            pltpu.sync_copy(dst_hbm.at[pl.ds(cbase, KI), :], didx)
            pltpu.sync_copy(mask_hbm.at[pl.ds(base, CHUNK)], mask_v)

            for j in range(KI):
                @pl.loop(0, IB, step=L)
                def _(k):
                    sidx[j, pl.ds(k, L)] = sidx[j, pl.ds(k, L)] + goff

            cps = [
                pltpu.async_copy(h_hbm.at[sidx.at[j]],
                                 rows_v.at[pl.ds(j * IB, IB), :], sem)
                for j in range(KI)
            ]
            for cp in cps:
                cp.wait()

            @pl.loop(0, CHUNK)
            def _(i):
                mb = plsc.load_gather(mask_v, [jnp.full((L,), i, jnp.int32)])
                rows_v[i, :] = mb * rows_v[i, :]

            for j in range(KI):
                pltpu.sync_copy(rows_v.at[pl.ds(j * IB, IB), :],
                                acc_sh.at[didx.at[j]], add=True)

        plsc.subcore_barrier()
        pltpu.sync_copy(acc_sh.at[pl.ds(row0, rows_sub), :],
                        out_hbm.at[cid, pl.ds(row0, rows_sub), :])

    return phase_b


def _prep_call(ef_p, mask_p, E, EP):
    BE = 32768
    grid = EP // BE

    def prep(ef_ref, m_ref, o_ref):
        i = pl.program_id(0)
        m = m_ref[...]
        valid = ((lax.broadcasted_iota(jnp.int32, (BE, 1), 0) + i * BE)
                 < E).astype(jnp.float32)
        o_ref[...] = jnp.concatenate(
            [ef_ref[...] * m,
             jnp.zeros((BE, 7), jnp.float32),
             valid,
             jnp.zeros((BE, 3), jnp.float32),
             m], axis=1)

    return pl.pallas_call(
        prep,
        grid=(grid,),
        in_specs=[pl.BlockSpec((BE, 4), lambda i: (i, 0)),
                  pl.BlockSpec((BE, 1), lambda i: (i, 0))],
        out_specs=pl.BlockSpec((BE, 16), lambda i: (i, 0)),
        out_shape=jax.ShapeDtypeStruct((EP, 16), jnp.float32),
    )(ef_p, mask_p)


def _dense1_call(pA, W1p, b1, W2, b2, N):
    BN = 5000
    grid = N // BN

    def dense1(pa_ref, w1_ref, b1_ref, w2_ref, b2_ref, h_ref, dinv_ref):
        s = pa_ref[0] + pa_ref[1]
        deg = jnp.maximum(s[:, 11:12], 1.0)
        dinv = 1.0 / deg
        hn = s * dinv
        z = jnp.dot(hn, w1_ref[...], preferred_element_type=jnp.float32)
        z = jnp.maximum(z + b1_ref[...], 0.0)
        h = jnp.dot(z, w2_ref[...], preferred_element_type=jnp.float32)
        h = jnp.maximum(h + b2_ref[...], 0.0)
        h_ref[0] = h[:, :16]
        h_ref[1] = h[:, 16:]
        dinv_ref[...] = dinv

    return pl.pallas_call(
        dense1,
        grid=(grid,),
        in_specs=[pl.BlockSpec((2, BN, 16), lambda i: (0, i, 0)),
                  pl.BlockSpec((16, 32), lambda i: (0, 0)),
                  pl.BlockSpec((1, 32), lambda i: (0, 0)),
                  pl.BlockSpec((32, 32), lambda i: (0, 0)),
                  pl.BlockSpec((1, 32), lambda i: (0, 0))],
        out_specs=[pl.BlockSpec((2, BN, 16), lambda i: (0, i, 0)),
                   pl.BlockSpec((BN, 1), lambda i: (i, 0))],
        out_shape=[jax.ShapeDtypeStruct((2, N, 16), jnp.float32),
                   jax.ShapeDtypeStruct((N, 1), jnp.float32)],
    )(pA, W1p, b1, W2, b2)


def _dense2_call(s2, dinv, W3, b3, W4, b4, W5, b5, W6, b6, N):
    BN = 5000
    grid = N // BN

    def dense2(s2_ref, dinv_ref, w3_ref, b3_ref, w4_ref, b4_ref,
               w5_ref, b5_ref, w6_ref, b6_ref, out_ref, mscr):
        i = pl.program_id(0)
        hcat = jnp.concatenate([s2_ref[0], s2_ref[1]], axis=1)
        hn = hcat * dinv_ref[...]
        h = jnp.dot(hn, w3_ref[...], preferred_element_type=jnp.float32)
        h = jnp.maximum(h + b3_ref[...], 0.0)
        h = jnp.dot(h, w4_ref[...], preferred_element_type=jnp.float32)
        h = jnp.maximum(h + b4_ref[...], 0.0)
        bmax = jnp.max(h, axis=0, keepdims=True)

        @pl.when(i == 0)
        def _():
            mscr[...] = bmax

        @pl.when(i > 0)
        def _():
            mscr[...] = jnp.maximum(mscr[...], bmax)

        @pl.when(i == grid - 1)
        def _():
            pooled = mscr[...]
            t = jnp.dot(pooled, w5_ref[...], preferred_element_type=jnp.float32)
            t = jnp.maximum(t + b5_ref[...], 0.0)
            logits = jnp.dot(t, w6_ref[...],
                             preferred_element_type=jnp.float32) + b6_ref[...]
            z = logits - jnp.max(logits, axis=1, keepdims=True)
            ez = jnp.exp(z)
            out_ref[...] = ez / jnp.sum(ez, axis=1, keepdims=True)

    return pl.pallas_call(
        dense2,
        grid=(grid,),
        in_specs=[pl.BlockSpec((2, BN, 16), lambda i: (0, i, 0)),
                  pl.BlockSpec((BN, 1), lambda i: (i, 0)),
                  pl.BlockSpec((32, 32), lambda i: (0, 0)),
                  pl.BlockSpec((1, 32), lambda i: (0, 0)),
                  pl.BlockSpec((32, 32), lambda i: (0, 0)),
                  pl.BlockSpec((1, 32), lambda i: (0, 0)),
                  pl.BlockSpec((32, 16), lambda i: (0, 0)),
                  pl.BlockSpec((1, 16), lambda i: (0, 0)),
                  pl.BlockSpec((16, 2), lambda i: (0, 0)),
                  pl.BlockSpec((1, 2), lambda i: (0, 0))],
        out_specs=pl.BlockSpec((1, 2), lambda i: (0, 0)),
        out_shape=jax.ShapeDtypeStruct((1, 2), jnp.float32),
        scratch_shapes=[pltpu.VMEM((1, 32), jnp.float32)],
    )(s2, dinv, W3, b3, W4, b4, W5, b5, W6, b6)


def kernel(node_feat, edge_feat, edge_mask, edge_index,
           W1, b1, W2, b2, W3, b3, W4, b4, W5, b5, W6, b6):
    N, FN = node_feat.shape
    E = edge_index.shape[1]
    EP = _round_up(E, NC * NS * CHUNK)

    src_p = jnp.pad(edge_index[0], (0, EP - E))
    dst_p = jnp.pad(edge_index[1], (0, EP - E))
    mask_p = jnp.pad(edge_mask, ((0, EP - E), (0, 0)))
    ef_p = jnp.pad(edge_feat, ((0, EP - E), (0, 0)))
    src2 = src_p.reshape(EP // IB, IB)
    dst2 = dst_p.reshape(EP // IB, IB)
    nf16 = jnp.pad(node_feat, ((0, 0), (4, 16 - 4 - FN)))
    W1p = jnp.pad(W1, ((0, 16 - W1.shape[0]), (0, 0)))

    pe16 = _prep_call(ef_p, mask_p, E, EP)
    pA = _phase_a(N, EP)(src2, dst2, pe16, nf16)
    h2, dinv = _dense1_call(pA, W1p, b1.reshape(1, -1), W2,
                            b2.reshape(1, -1), N)
    h2flat = h2.reshape(NC * N, 16)
    s2 = _phase_b(N, EP)(src2, dst2, mask_p.reshape(EP), h2flat)
    pred = _dense2_call(s2, dinv, W3, b3.reshape(1, -1), W4,
                        b4.reshape(1, -1), W5, b5.reshape(1, -1),
                        W6, b6.reshape(1, -1), N)
    return pred


# speedup vs baseline: 2.0071x; 2.0071x over previous
import dataclasses
import functools

import jax
import jax.numpy as jnp
from jax import lax
from jax.experimental import pallas as pl
from jax.experimental.pallas import tpu as pltpu
from jax.experimental.pallas import tpu_sc as plsc

NC = 2
NS = 16
L = 16
IB = 128
CHUNK = 512
KI = CHUNK // IB


def _round_up(x, m):
    return (x + m - 1) // m * m


def _zero_block(rows_sub):
    for cand in range(min(512, rows_sub), 7, -1):
        if rows_sub % cand == 0 and cand % 8 == 0:
            return cand
    return 8


def _sc_params():
    cp = pltpu.CompilerParams()
    fields = pltpu.CompilerParams.__dataclass_fields__
    if "needs_layout_passes" in fields:
        cp = dataclasses.replace(cp, needs_layout_passes=False)
    if "use_tc_tiling_on_sc" in fields:
        cp = dataclasses.replace(cp, use_tc_tiling_on_sc=False)
    return cp


def _phase_a(NP, EP):
    ea = EP // (NC * NS)
    nchunks = ea // CHUNK
    rows_sub = NP // NS
    zb = _zero_block(rows_sub)
    nz = rows_sub // zb

    mesh = plsc.VectorSubcoreMesh(core_axis_name="c", subcore_axis_name="s")

    @functools.partial(
        pl.kernel,
        out_type=jax.ShapeDtypeStruct((NC, NP, 16), jnp.float32),
        mesh=mesh,
        scratch_types=(
            [pltpu.VMEM((IB,), jnp.int32) for _ in range(KI)]
            + [pltpu.VMEM((IB,), jnp.int32) for _ in range(KI)]
            + [pltpu.VMEM((IB, 16), jnp.float32) for _ in range(KI)]
            + [
                pltpu.VMEM((CHUNK, 16), jnp.float32),
                pltpu.VMEM((zb, 16), jnp.float32),
                pltpu.VMEM_SHARED((NP, 16), jnp.float32),
                pltpu.SemaphoreType.DMA,
            ]
        ),
        compiler_params=_sc_params(),
    )
    def phase_a(src_hbm, dst_hbm, pe_hbm, nf_hbm, out_hbm, *refs):
        sidx = refs[0:KI]
        didx = refs[KI:2 * KI]
        rows = refs[2 * KI:3 * KI]
        pe_v, zb_v, acc_sh, sem = refs[3 * KI:]
        cid = lax.axis_index("c")
        sid = lax.axis_index("s")
        wid = sid * NC + cid

        @pl.loop(0, zb)
        def _(i):
            zb_v[i, :] = jnp.zeros((16,), jnp.float32)

        row0 = sid * rows_sub

        @pl.loop(0, nz)
        def _(k):
            pltpu.sync_copy(zb_v.at[pl.ds(0, zb), :],
                            acc_sh.at[pl.ds(row0 + k * zb, zb), :])

        plsc.subcore_barrier()

        base0 = wid * ea

        @pl.loop(0, nchunks)
        def _(ci):
            base = pl.multiple_of(base0 + ci * CHUNK, IB)
            pltpu.sync_copy(pe_hbm.at[pl.ds(base, CHUNK), :], pe_v)
            for j in range(KI):
                pltpu.sync_copy(src_hbm.at[pl.ds(base + j * IB, IB)],
                                sidx[j])
                pltpu.sync_copy(dst_hbm.at[pl.ds(base + j * IB, IB)],
                                didx[j])
            cps = [
                pltpu.async_copy(nf_hbm.at[sidx[j]], rows[j], sem)
                for j in range(KI)
            ]
            for cp in cps:
                cp.wait()

            for j in range(KI):
                @pl.loop(0, IB)
                def _(i):
                    mrow = jnp.full((L,), j * IB + i, jnp.int32)
                    mcol = jnp.full((L,), 15, jnp.int32)
                    mb = plsc.load_gather(pe_v, [mrow, mcol])
                    rows[j][i, :] = (mb * rows[j][i, :]
                                     + pe_v[j * IB + i, :])

            for j in range(KI):
                pltpu.sync_copy(rows[j], acc_sh.at[didx[j]], add=True)

        plsc.subcore_barrier()
        pltpu.sync_copy(acc_sh.at[pl.ds(row0, rows_sub), :],
                        out_hbm.at[cid, pl.ds(row0, rows_sub), :])

    return phase_a


def _phase_b(NP, EP):
    eb = EP // NS
    nchunks = eb // CHUNK
    rows_sub = NP // NS
    zb = _zero_block(rows_sub)
    nz = rows_sub // zb

    mesh = plsc.VectorSubcoreMesh(core_axis_name="c", subcore_axis_name="s")

    @functools.partial(
        pl.kernel,
        out_type=jax.ShapeDtypeStruct((NC, NP, 16), jnp.float32),
        mesh=mesh,
        scratch_types=(
            [pltpu.VMEM((IB,), jnp.int32) for _ in range(KI)]
            + [pltpu.VMEM((IB,), jnp.int32) for _ in range(KI)]
            + [pltpu.VMEM((IB, 16), jnp.float32) for _ in range(KI)]
            + [
                pltpu.VMEM((CHUNK,), jnp.float32),
                pltpu.VMEM((zb, 16), jnp.float32),
                pltpu.VMEM_SHARED((NP, 16), jnp.float32),
                pltpu.SemaphoreType.DMA,
            ]
        ),
        compiler_params=_sc_params(),
    )
    def phase_b(src_hbm, dst_hbm, mask_hbm, h_hbm, out_hbm, *refs):
        sidx = refs[0:KI]
        didx = refs[KI:2 * KI]
        rows = refs[2 * KI:3 * KI]
        mask_v, zb_v, acc_sh, sem = refs[3 * KI:]
        cid = lax.axis_index("c")
        sid = lax.axis_index("s")

        @pl.loop(0, zb)
        def _(i):
            zb_v[i, :] = jnp.zeros((16,), jnp.float32)

        row0 = sid * rows_sub

        @pl.loop(0, nz)
        def _(k):
            pltpu.sync_copy(zb_v.at[pl.ds(0, zb), :],
                            acc_sh.at[pl.ds(row0 + k * zb, zb), :])

        plsc.subcore_barrier()

        base0 = sid * eb
        goff = cid * NP

        @pl.loop(0, nchunks)
        def _(ci):
            base = pl.multiple_of(base0 + ci * CHUNK, IB)
            pltpu.sync_copy(mask_hbm.at[pl.ds(base, CHUNK)], mask_v)
            for j in range(KI):
                pltpu.sync_copy(src_hbm.at[pl.ds(base + j * IB, IB)],
                                sidx[j])
                pltpu.sync_copy(dst_hbm.at[pl.ds(base + j * IB, IB)],
                                didx[j])

            for j in range(KI):
                @pl.loop(0, IB, step=L)
                def _(k):
                    sidx[j][pl.ds(k, L)] = sidx[j][pl.ds(k, L)] + goff

            cps = [
                pltpu.async_copy(h_hbm.at[sidx[j]], rows[j], sem)
                for j in range(KI)
            ]
            for cp in cps:
                cp.wait()

            for j in range(KI):
                @pl.loop(0, IB)
                def _(i):
                    mb = plsc.load_gather(
                        mask_v, [jnp.full((L,), j * IB + i, jnp.int32)])
                    rows[j][i, :] = mb * rows[j][i, :]

            for j in range(KI):
                pltpu.sync_copy(rows[j], acc_sh.at[didx[j]], add=True)

        plsc.subcore_barrier()
        pltpu.sync_copy(acc_sh.at[pl.ds(row0, rows_sub), :],
                        out_hbm.at[cid, pl.ds(row0, rows_sub), :])

    return phase_b


def _prep_call(ef_p, mask_p, E, EP):
    BE = 2048
    grid = EP // BE

    def prep(ef_ref, m_ref, o_ref):
        i = pl.program_id(0)
        m = m_ref[...]
        valid = ((lax.broadcasted_iota(jnp.int32, (BE, 1), 0) + i * BE)
                 < E).astype(jnp.float32)
        o_ref[...] = jnp.concatenate(
            [ef_ref[...] * m,
             jnp.zeros((BE, 7), jnp.float32),
             valid,
             jnp.zeros((BE, 3), jnp.float32),
             m], axis=1)

    return pl.pallas_call(
        prep,
        grid=(grid,),
        in_specs=[pl.BlockSpec((BE, 4), lambda i: (i, 0)),
                  pl.BlockSpec((BE, 1), lambda i: (i, 0))],
        out_specs=pl.BlockSpec((BE, 16), lambda i: (i, 0)),
        out_shape=jax.ShapeDtypeStruct((EP, 16), jnp.float32),
    )(ef_p, mask_p)


def _dense1_call(pA, W1p, b1, W2, b2, NP):
    BN = NP // 32
    grid = NP // BN

    def dense1(pa_ref, w1_ref, b1_ref, w2_ref, b2_ref, h_ref, dinv_ref):
        s = pa_ref[0] + pa_ref[1]
        deg = jnp.maximum(s[:, 11:12], 1.0)
        dinv = 1.0 / deg
        hn = s * dinv
        z = jnp.dot(hn, w1_ref[...], preferred_element_type=jnp.float32)
        z = jnp.maximum(z + b1_ref[...], 0.0)
        h = jnp.dot(z, w2_ref[...], preferred_element_type=jnp.float32)
        h = jnp.maximum(h + b2_ref[...], 0.0)
        h_ref[0] = h[:, :16]
        h_ref[1] = h[:, 16:]
        dinv_ref[...] = dinv

    return pl.pallas_call(
        dense1,
        grid=(grid,),
        in_specs=[pl.BlockSpec((2, BN, 16), lambda i: (0, i, 0)),
                  pl.BlockSpec((16, 32), lambda i: (0, 0)),
                  pl.BlockSpec((1, 32), lambda i: (0, 0)),
                  pl.BlockSpec((32, 32), lambda i: (0, 0)),
                  pl.BlockSpec((1, 32), lambda i: (0, 0))],
        out_specs=[pl.BlockSpec((2, BN, 16), lambda i: (0, i, 0)),
                   pl.BlockSpec((BN, 1), lambda i: (i, 0))],
        out_shape=[jax.ShapeDtypeStruct((2, NP, 16), jnp.float32),
                   jax.ShapeDtypeStruct((NP, 1), jnp.float32)],
    )(pA, W1p, b1, W2, b2)


def _dense2_call(s2, dinv, W3, b3, W4, b4, W5, b5, W6, b6, N, NP):
    BN = NP // 32
    grid = NP // BN

    def dense2(s2_ref, dinv_ref, w3_ref, b3_ref, w4_ref, b4_ref,
               w5_ref, b5_ref, w6_ref, b6_ref, out_ref, mscr):
        i = pl.program_id(0)
        hcat = jnp.concatenate([s2_ref[0], s2_ref[1]], axis=1)
        hn = hcat * dinv_ref[...]
        h = jnp.dot(hn, w3_ref[...], preferred_element_type=jnp.float32)
        h = jnp.maximum(h + b3_ref[...], 0.0)
        h = jnp.dot(h, w4_ref[...], preferred_element_type=jnp.float32)
        h = jnp.maximum(h + b4_ref[...], 0.0)
        valid = (lax.broadcasted_iota(jnp.int32, (BN, 1), 0) + i * BN) < N
        h = jnp.where(valid, h, -jnp.inf)
        bmax = jnp.max(h, axis=0, keepdims=True)

        @pl.when(i == 0)
        def _():
            mscr[...] = bmax

        @pl.when(i > 0)
        def _():
            mscr[...] = jnp.maximum(mscr[...], bmax)

        @pl.when(i == grid - 1)
        def _():
            pooled = mscr[...]
            t = jnp.dot(pooled, w5_ref[...], preferred_element_type=jnp.float32)
            t = jnp.maximum(t + b5_ref[...], 0.0)
            logits = jnp.dot(t, w6_ref[...],
                             preferred_element_type=jnp.float32) + b6_ref[...]
            z = logits - jnp.max(logits, axis=1, keepdims=True)
            ez = jnp.exp(z)
            out_ref[...] = ez / jnp.sum(ez, axis=1, keepdims=True)

    return pl.pallas_call(
        dense2,
        grid=(grid,),
        in_specs=[pl.BlockSpec((2, BN, 16), lambda i: (0, i, 0)),
                  pl.BlockSpec((BN, 1), lambda i: (i, 0)),
                  pl.BlockSpec((32, 32), lambda i: (0, 0)),
                  pl.BlockSpec((1, 32), lambda i: (0, 0)),
                  pl.BlockSpec((32, 32), lambda i: (0, 0)),
                  pl.BlockSpec((1, 32), lambda i: (0, 0)),
                  pl.BlockSpec((32, 16), lambda i: (0, 0)),
                  pl.BlockSpec((1, 16), lambda i: (0, 0)),
                  pl.BlockSpec((16, 2), lambda i: (0, 0)),
                  pl.BlockSpec((1, 2), lambda i: (0, 0))],
        out_specs=pl.BlockSpec((1, 2), lambda i: (0, 0)),
        out_shape=jax.ShapeDtypeStruct((1, 2), jnp.float32),
        scratch_shapes=[pltpu.VMEM((1, 32), jnp.float32)],
    )(s2, dinv, W3, b3, W4, b4, W5, b5, W6, b6)


def kernel(node_feat, edge_feat, edge_mask, edge_index,
           W1, b1, W2, b2, W3, b3, W4, b4, W5, b5, W6, b6):
    N, FN = node_feat.shape
    NP = _round_up(N, 128)
    E = edge_index.shape[1]
    EP = _round_up(E, NC * NS * CHUNK)

    src_p = jnp.pad(edge_index[0], (0, EP - E))
    dst_p = jnp.pad(edge_index[1], (0, EP - E))
    mask_p = jnp.pad(edge_mask, ((0, EP - E), (0, 0)))
    ef_p = jnp.pad(edge_feat, ((0, EP - E), (0, 0)))
    src2 = src_p
    dst2 = dst_p
    nf16 = jnp.pad(node_feat, ((0, 0), (4, 16 - 4 - FN)))
    W1p = jnp.pad(W1, ((0, 16 - W1.shape[0]), (0, 0)))

    pe16 = _prep_call(ef_p, mask_p, E, EP)
    pA = _phase_a(NP, EP)(src2, dst2, pe16, nf16)
    h2, dinv = _dense1_call(pA, W1p, b1.reshape(1, -1), W2,
                            b2.reshape(1, -1), NP)
    h2flat = h2.reshape(NC * NP, 16)
    s2 = _phase_b(NP, EP)(src2, dst2, mask_p.reshape(EP), h2flat)
    pred = _dense2_call(s2, dinv, W3, b3.reshape(1, -1), W4,
                        b4.reshape(1, -1), W5, b5.reshape(1, -1),
                        W6, b6.reshape(1, -1), N, NP)
    return pred


# speedup vs baseline: 2.0311x; 1.0120x over previous
import dataclasses
import functools

import jax
import jax.numpy as jnp
from jax import lax
from jax.experimental import pallas as pl
from jax.experimental.pallas import tpu as pltpu
from jax.experimental.pallas import tpu_sc as plsc

NC = 2
NS = 16
L = 16
IB = 128
CHUNK = 512
KI = CHUNK // IB


def _round_up(x, m):
    return (x + m - 1) // m * m


def _zero_block(rows_sub):
    for cand in range(min(512, rows_sub), 7, -1):
        if rows_sub % cand == 0 and cand % 8 == 0:
            return cand
    return 8


def _sc_params():
    cp = pltpu.CompilerParams()
    fields = pltpu.CompilerParams.__dataclass_fields__
    if "needs_layout_passes" in fields:
        cp = dataclasses.replace(cp, needs_layout_passes=False)
    if "use_tc_tiling_on_sc" in fields:
        cp = dataclasses.replace(cp, use_tc_tiling_on_sc=False)
    return cp


def _phase_a(NP, EP):
    ea = EP // (NC * NS)
    nchunks = ea // CHUNK
    rows_sub = NP // NS
    zb = _zero_block(rows_sub)
    nz = rows_sub // zb

    mesh = plsc.VectorSubcoreMesh(core_axis_name="c", subcore_axis_name="s")

    @functools.partial(
        pl.kernel,
        out_type=jax.ShapeDtypeStruct((NC, NP, 16), jnp.float32),
        mesh=mesh,
        scratch_types=(
            [pltpu.VMEM((IB,), jnp.int32) for _ in range(KI)]
            + [pltpu.VMEM((IB,), jnp.int32) for _ in range(KI)]
            + [pltpu.VMEM((IB, 16), jnp.float32) for _ in range(KI)]
            + [
                pltpu.VMEM((CHUNK, 16), jnp.float32),
                pltpu.VMEM((zb, 16), jnp.float32),
                pltpu.VMEM_SHARED((NP, 16), jnp.float32),
                pltpu.SemaphoreType.DMA,
            ]
        ),
        compiler_params=_sc_params(),
    )
    def phase_a(src_hbm, dst_hbm, pe_hbm, nf_hbm, out_hbm, *refs):
        sidx = refs[0:KI]
        didx = refs[KI:2 * KI]
        rows = refs[2 * KI:3 * KI]
        pe_v, zb_v, acc_sh, sem = refs[3 * KI:]
        cid = lax.axis_index("c")
        sid = lax.axis_index("s")
        wid = sid * NC + cid

        @pl.loop(0, zb)
        def _(i):
            zb_v[i, :] = jnp.zeros((16,), jnp.float32)

        row0 = sid * rows_sub

        @pl.loop(0, nz)
        def _(k):
            pltpu.sync_copy(zb_v.at[pl.ds(0, zb), :],
                            acc_sh.at[pl.ds(row0 + k * zb, zb), :])

        plsc.subcore_barrier()

        base0 = wid * ea

        @pl.loop(0, nchunks)
        def _(ci):
            base = pl.multiple_of(base0 + ci * CHUNK, IB)
            pltpu.sync_copy(pe_hbm.at[pl.ds(base, CHUNK), :], pe_v)
            for j in range(KI):
                pltpu.sync_copy(src_hbm.at[pl.ds(base + j * IB, IB)],
                                sidx[j])
                pltpu.sync_copy(dst_hbm.at[pl.ds(base + j * IB, IB)],
                                didx[j])
            cps = [
                pltpu.async_copy(nf_hbm.at[sidx[j]], rows[j], sem)
                for j in range(KI)
            ]
            for cp in cps:
                cp.wait()

            for j in range(KI):
                @pl.loop(0, IB, unroll=8)
                def _(i):
                    mrow = jnp.full((L,), j * IB + i, jnp.int32)
                    mcol = jnp.full((L,), 15, jnp.int32)
                    mb = plsc.load_gather(pe_v, [mrow, mcol])
                    rows[j][i, :] = (mb * rows[j][i, :]
                                     + pe_v[j * IB + i, :])

            for j in range(KI):
                pltpu.sync_copy(rows[j], acc_sh.at[didx[j]], add=True)

        plsc.subcore_barrier()
        pltpu.sync_copy(acc_sh.at[pl.ds(row0, rows_sub), :],
                        out_hbm.at[cid, pl.ds(row0, rows_sub), :])

    return phase_a


def _phase_b(NP, EP):
    eb = EP // NS
    nchunks = eb // CHUNK
    rows_sub = NP // NS
    zb = _zero_block(rows_sub)
    nz = rows_sub // zb

    mesh = plsc.VectorSubcoreMesh(core_axis_name="c", subcore_axis_name="s")

    @functools.partial(
        pl.kernel,
        out_type=jax.ShapeDtypeStruct((NC, NP, 16), jnp.float32),
        mesh=mesh,
        scratch_types=(
            [pltpu.VMEM((IB,), jnp.int32) for _ in range(KI)]
            + [pltpu.VMEM((IB,), jnp.int32) for _ in range(KI)]
            + [pltpu.VMEM((IB, 16), jnp.float32) for _ in range(KI)]
            + [
                pltpu.VMEM((CHUNK,), jnp.float32),
                pltpu.VMEM((zb, 16), jnp.float32),
                pltpu.VMEM_SHARED((NP, 16), jnp.float32),
                pltpu.SemaphoreType.DMA,
            ]
        ),
        compiler_params=_sc_params(),
    )
    def phase_b(src_hbm, dst_hbm, mask_hbm, h_hbm, out_hbm, *refs):
        sidx = refs[0:KI]
        didx = refs[KI:2 * KI]
        rows = refs[2 * KI:3 * KI]
        mask_v, zb_v, acc_sh, sem = refs[3 * KI:]
        cid = lax.axis_index("c")
        sid = lax.axis_index("s")

        @pl.loop(0, zb)
        def _(i):
            zb_v[i, :] = jnp.zeros((16,), jnp.float32)

        row0 = sid * rows_sub

        @pl.loop(0, nz)
        def _(k):
            pltpu.sync_copy(zb_v.at[pl.ds(0, zb), :],
                            acc_sh.at[pl.ds(row0 + k * zb, zb), :])

        plsc.subcore_barrier()

        base0 = sid * eb
        goff = cid * NP

        @pl.loop(0, nchunks)
        def _(ci):
            base = pl.multiple_of(base0 + ci * CHUNK, IB)
            pltpu.sync_copy(mask_hbm.at[pl.ds(base, CHUNK)], mask_v)
            for j in range(KI):
                pltpu.sync_copy(src_hbm.at[pl.ds(base + j * IB, IB)],
                                sidx[j])
                pltpu.sync_copy(dst_hbm.at[pl.ds(base + j * IB, IB)],
                                didx[j])

            for j in range(KI):
                @pl.loop(0, IB, step=L)
                def _(k):
                    sidx[j][pl.ds(k, L)] = sidx[j][pl.ds(k, L)] + goff

            cps = [
                pltpu.async_copy(h_hbm.at[sidx[j]], rows[j], sem)
                for j in range(KI)
            ]
            for cp in cps:
                cp.wait()

            for j in range(KI):
                @pl.loop(0, IB, unroll=8)
                def _(i):
                    mb = plsc.load_gather(
                        mask_v, [jnp.full((L,), j * IB + i, jnp.int32)])
                    rows[j][i, :] = mb * rows[j][i, :]

            for j in range(KI):
                pltpu.sync_copy(rows[j], acc_sh.at[didx[j]], add=True)

        plsc.subcore_barrier()
        pltpu.sync_copy(acc_sh.at[pl.ds(row0, rows_sub), :],
                        out_hbm.at[cid, pl.ds(row0, rows_sub), :])

    return phase_b


def _prep_call(ef_p, mask_p, E, EP):
    BE = 2048
    grid = EP // BE

    def prep(ef_ref, m_ref, o_ref):
        i = pl.program_id(0)
        m = m_ref[...]
        valid = ((lax.broadcasted_iota(jnp.int32, (BE, 1), 0) + i * BE)
                 < E).astype(jnp.float32)
        o_ref[...] = jnp.concatenate(
            [ef_ref[...] * m,
             jnp.zeros((BE, 7), jnp.float32),
             valid,
             jnp.zeros((BE, 3), jnp.float32),
             m], axis=1)

    return pl.pallas_call(
        prep,
        grid=(grid,),
        in_specs=[pl.BlockSpec((BE, 4), lambda i: (i, 0)),
                  pl.BlockSpec((BE, 1), lambda i: (i, 0))],
        out_specs=pl.BlockSpec((BE, 16), lambda i: (i, 0)),
        out_shape=jax.ShapeDtypeStruct((EP, 16), jnp.float32),
    )(ef_p, mask_p)


def _dense1_call(pA, W1p, b1, W2, b2, NP):
    BN = NP // 32
    grid = NP // BN

    def dense1(pa_ref, w1_ref, b1_ref, w2_ref, b2_ref, h_ref, dinv_ref):
        s = pa_ref[0] + pa_ref[1]
        deg = jnp.maximum(s[:, 11:12], 1.0)
        dinv = 1.0 / deg
        hn = s * dinv
        z = jnp.dot(hn, w1_ref[...], preferred_element_type=jnp.float32)
        z = jnp.maximum(z + b1_ref[...], 0.0)
        h = jnp.dot(z, w2_ref[...], preferred_element_type=jnp.float32)
        h = jnp.maximum(h + b2_ref[...], 0.0)
        h_ref[0] = h[:, :16]
        h_ref[1] = h[:, 16:]
        dinv_ref[...] = dinv

    return pl.pallas_call(
        dense1,
        grid=(grid,),
        in_specs=[pl.BlockSpec((2, BN, 16), lambda i: (0, i, 0)),
                  pl.BlockSpec((16, 32), lambda i: (0, 0)),
                  pl.BlockSpec((1, 32), lambda i: (0, 0)),
                  pl.BlockSpec((32, 32), lambda i: (0, 0)),
                  pl.BlockSpec((1, 32), lambda i: (0, 0))],
        out_specs=[pl.BlockSpec((2, BN, 16), lambda i: (0, i, 0)),
                   pl.BlockSpec((BN, 1), lambda i: (i, 0))],
        out_shape=[jax.ShapeDtypeStruct((2, NP, 16), jnp.float32),
                   jax.ShapeDtypeStruct((NP, 1), jnp.float32)],
    )(pA, W1p, b1, W2, b2)


def _dense2_call(s2, dinv, W3, b3, W4, b4, W5, b5, W6, b6, N, NP):
    BN = NP // 32
    grid = NP // BN

    def dense2(s2_ref, dinv_ref, w3_ref, b3_ref, w4_ref, b4_ref,
               w5_ref, b5_ref, w6_ref, b6_ref, out_ref, mscr):
        i = pl.program_id(0)
        hcat = jnp.concatenate([s2_ref[0], s2_ref[1]], axis=1)
        hn = hcat * dinv_ref[...]
        h = jnp.dot(hn, w3_ref[...], preferred_element_type=jnp.float32)
        h = jnp.maximum(h + b3_ref[...], 0.0)
        h = jnp.dot(h, w4_ref[...], preferred_element_type=jnp.float32)
        h = jnp.maximum(h + b4_ref[...], 0.0)
        valid = (lax.broadcasted_iota(jnp.int32, (BN, 1), 0) + i * BN) < N
        h = jnp.where(valid, h, -jnp.inf)
        bmax = jnp.max(h, axis=0, keepdims=True)

        @pl.when(i == 0)
        def _():
            mscr[...] = bmax

        @pl.when(i > 0)
        def _():
            mscr[...] = jnp.maximum(mscr[...], bmax)

        @pl.when(i == grid - 1)
        def _():
            pooled = mscr[...]
            t = jnp.dot(pooled, w5_ref[...], preferred_element_type=jnp.float32)
            t = jnp.maximum(t + b5_ref[...], 0.0)
            logits = jnp.dot(t, w6_ref[...],
                             preferred_element_type=jnp.float32) + b6_ref[...]
            z = logits - jnp.max(logits, axis=1, keepdims=True)
            ez = jnp.exp(z)
            out_ref[...] = ez / jnp.sum(ez, axis=1, keepdims=True)

    return pl.pallas_call(
        dense2,
        grid=(grid,),
        in_specs=[pl.BlockSpec((2, BN, 16), lambda i: (0, i, 0)),
                  pl.BlockSpec((BN, 1), lambda i: (i, 0)),
                  pl.BlockSpec((32, 32), lambda i: (0, 0)),
                  pl.BlockSpec((1, 32), lambda i: (0, 0)),
                  pl.BlockSpec((32, 32), lambda i: (0, 0)),
                  pl.BlockSpec((1, 32), lambda i: (0, 0)),
                  pl.BlockSpec((32, 16), lambda i: (0, 0)),
                  pl.BlockSpec((1, 16), lambda i: (0, 0)),
                  pl.BlockSpec((16, 2), lambda i: (0, 0)),
                  pl.BlockSpec((1, 2), lambda i: (0, 0))],
        out_specs=pl.BlockSpec((1, 2), lambda i: (0, 0)),
        out_shape=jax.ShapeDtypeStruct((1, 2), jnp.float32),
        scratch_shapes=[pltpu.VMEM((1, 32), jnp.float32)],
    )(s2, dinv, W3, b3, W4, b4, W5, b5, W6, b6)


def kernel(node_feat, edge_feat, edge_mask, edge_index,
           W1, b1, W2, b2, W3, b3, W4, b4, W5, b5, W6, b6):
    N, FN = node_feat.shape
    NP = _round_up(N, 128)
    E = edge_index.shape[1]
    EP = _round_up(E, NC * NS * CHUNK)

    src_p = jnp.pad(edge_index[0], (0, EP - E))
    dst_p = jnp.pad(edge_index[1], (0, EP - E))
    mask_p = jnp.pad(edge_mask, ((0, EP - E), (0, 0)))
    ef_p = jnp.pad(edge_feat, ((0, EP - E), (0, 0)))
    src2 = src_p
    dst2 = dst_p
    nf16 = jnp.pad(node_feat, ((0, 0), (4, 16 - 4 - FN)))
    W1p = jnp.pad(W1, ((0, 16 - W1.shape[0]), (0, 0)))

    pe16 = _prep_call(ef_p, mask_p, E, EP)
    pA = _phase_a(NP, EP)(src2, dst2, pe16, nf16)
    h2, dinv = _dense1_call(pA, W1p, b1.reshape(1, -1), W2,
                            b2.reshape(1, -1), NP)
    h2flat = h2.reshape(NC * NP, 16)
    s2 = _phase_b(NP, EP)(src2, dst2, mask_p.reshape(EP), h2flat)
    pred = _dense2_call(s2, dinv, W3, b3.reshape(1, -1), W4,
                        b4.reshape(1, -1), W5, b5.reshape(1, -1),
                        W6, b6.reshape(1, -1), N, NP)
    return pred
